# initial kernel scaffold (unmeasured)
import jax
import jax.numpy as jnp
from jax import lax
from jax.experimental import pallas as pl
from jax.experimental.pallas import tpu as pltpu


def kernel(
    x,
):
    def body(*refs):
        pass

    out_shape = jax.ShapeDtypeStruct(..., jnp.float32)
    return pl.pallas_call(body, out_shape=out_shape)(...)


# baseline (device time: 29553 ns/iter reference)
import jax
import jax.numpy as jnp
from jax import lax
from jax.experimental import pallas as pl
from jax.experimental.pallas import tpu as pltpu


def kernel(x):
    m_per, n = x.shape

    def body(x_ref, out_ref, send_sem, recv_sem):
        my_x = lax.axis_index("x")
        my_y = lax.axis_index("y")
        my_z = lax.axis_index("z")
        partner = (1 - my_x, my_y, my_z)

        barrier_sem = pltpu.get_barrier_semaphore()
        pl.semaphore_signal(
            barrier_sem, inc=1,
            device_id=partner, device_id_type=pl.DeviceIdType.MESH,
        )
        pl.semaphore_wait(barrier_sem, 1)

        rdma = pltpu.make_async_remote_copy(
            src_ref=x_ref,
            dst_ref=out_ref.at[pl.ds(my_x * m_per, m_per), :],
            send_sem=send_sem,
            recv_sem=recv_sem,
            device_id=partner,
            device_id_type=pl.DeviceIdType.MESH,
        )
        rdma.start()

        out_ref[pl.ds(my_x * m_per, m_per), :] = x_ref[...]

        rdma.wait()

    return pl.pallas_call(
        body,
        out_shape=jax.ShapeDtypeStruct((2 * m_per, n), x.dtype),
        in_specs=[pl.BlockSpec(memory_space=pltpu.VMEM)],
        out_specs=pl.BlockSpec(memory_space=pltpu.VMEM),
        scratch_shapes=[
            pltpu.SemaphoreType.DMA,
            pltpu.SemaphoreType.DMA,
        ],
        compiler_params=pltpu.CompilerParams(collective_id=0),
    )(x)


# device time: 22416 ns/iter; 1.3184x vs baseline; 1.3184x over previous
import jax
import jax.numpy as jnp
from jax import lax
from jax.experimental import pallas as pl
from jax.experimental.pallas import tpu as pltpu

K = 8


def kernel(x):
    m_per, n = x.shape
    half = m_per // 2
    sc = half // K

    def body(x_ref, out_ref, sends_x, recvs_x, sends_y, recvs_y):
        my_x = lax.axis_index("x")
        my_y = lax.axis_index("y")
        my_z = lax.axis_index("z")
        other_x = 1 - my_x
        p = my_y % 2
        partner_x = (other_x, my_y, my_z)
        partner_y = (my_x, my_y ^ 1, my_z)

        barrier_sem = pltpu.get_barrier_semaphore()
        for nbr in (partner_x, partner_y):
            pl.semaphore_signal(
                barrier_sem, inc=1,
                device_id=nbr, device_id_type=pl.DeviceIdType.MESH,
            )
        pl.semaphore_wait(barrier_sem, 2)

        send_base = p * half
        x_dst_base = my_x * m_per + p * half
        r1_base = other_x * m_per + p * half

        rdma_x = []
        for i in range(K):
            r = pltpu.make_async_remote_copy(
                src_ref=x_ref.at[pl.ds(send_base + i * sc, sc), :],
                dst_ref=out_ref.at[pl.ds(x_dst_base + i * sc, sc), :],
                send_sem=sends_x.at[i],
                recv_sem=recvs_x.at[i],
                device_id=partner_x,
                device_id_type=pl.DeviceIdType.MESH,
            )
            r.start()
            rdma_x.append(r)

        out_ref[pl.ds(my_x * m_per, m_per), :] = x_ref[...]

        rdma_y = []
        for i in range(K):
            rdma_x[i].wait_recv()
            r = pltpu.make_async_remote_copy(
                src_ref=out_ref.at[pl.ds(r1_base + i * sc, sc), :],
                dst_ref=out_ref.at[pl.ds(r1_base + i * sc, sc), :],
                send_sem=sends_y.at[i],
                recv_sem=recvs_y.at[i],
                device_id=partner_y,
                device_id_type=pl.DeviceIdType.MESH,
            )
            r.start()
            rdma_y.append(r)

        for i in range(K):
            rdma_x[i].wait_send()
            rdma_y[i].wait_send()
            rdma_y[i].wait_recv()

    return pl.pallas_call(
        body,
        out_shape=jax.ShapeDtypeStruct((2 * m_per, n), x.dtype),
        in_specs=[pl.BlockSpec(memory_space=pltpu.VMEM)],
        out_specs=pl.BlockSpec(memory_space=pltpu.VMEM),
        scratch_shapes=[
            pltpu.SemaphoreType.DMA((K,)),
            pltpu.SemaphoreType.DMA((K,)),
            pltpu.SemaphoreType.DMA((K,)),
            pltpu.SemaphoreType.DMA((K,)),
        ],
        compiler_params=pltpu.CompilerParams(collective_id=0),
    )(x)


# device time: 19371 ns/iter; 1.5256x vs baseline; 1.1572x over previous
import jax
import jax.numpy as jnp
from jax import lax
from jax.experimental import pallas as pl
from jax.experimental.pallas import tpu as pltpu

QR = 256
CH = 32
NCH = QR // CH

RELAY_Y = ((2, 80, 16), (3, 96, 32), (4, 128, 32), (5, 160, 8))
RELAY_Z = ((5, 168, 24), (6, 192, 32), (7, 224, 32))
DX_ROWS = 80

MESH = pl.DeviceIdType.MESH


def kernel(x):
    m_per, n = x.shape

    def body(x_ref, out_ref, sx, rx_sem, sy, ry_sem, sz, rz_sem):
        my_x = lax.axis_index("x")
        my_y = lax.axis_index("y")
        my_z = lax.axis_index("z")
        other_x = 1 - my_x
        p = my_y % 2
        q = my_z % 2
        partner_x = (other_x, my_y, my_z)
        partner_y = (my_x, my_y ^ 1, my_z)
        partner_z = (my_x, my_y, my_z ^ 1)

        qown = (2 * p + q) * QR
        qy = (2 * (1 - p) + q) * QR
        qz = (2 * p + (1 - q)) * QR
        qd = (2 * (1 - p) + (1 - q)) * QR
        own = my_x * m_per
        rem = other_x * m_per

        barrier_sem = pltpu.get_barrier_semaphore()
        for nbr in (partner_x, partner_y, partner_z):
            pl.semaphore_signal(
                barrier_sem, inc=1, device_id=nbr, device_id_type=MESH,
            )
        pl.semaphore_wait(barrier_sem, 3)

        def rdma(src, dst, send_sems, recv_sems, i, dev):
            return pltpu.make_async_remote_copy(
                src_ref=src, dst_ref=dst,
                send_sem=send_sems.at[i], recv_sem=recv_sems.at[i],
                device_id=dev, device_id_type=MESH,
            )

        rx = []
        for i in range(NCH):
            r = rdma(
                x_ref.at[pl.ds(qown + i * CH, CH), :],
                out_ref.at[pl.ds(own + qown + i * CH, CH), :],
                sx, rx_sem, i, partner_x,
            )
            r.start()
            rx.append(r)
        r = rdma(
            x_ref.at[pl.ds(qd, DX_ROWS), :],
            out_ref.at[pl.ds(own + qd, DX_ROWS), :],
            sx, rx_sem, NCH, partner_x,
        )
        r.start()
        rx.append(r)

        out_ref[pl.ds(own, m_per), :] = x_ref[...]

        ry, rz = [], []
        for i in range(NCH):
            rx[i].wait_recv()
            src = out_ref.at[pl.ds(rem + qown + i * CH, CH), :]
            r = rdma(src, src, sy, ry_sem, i, partner_y)
            r.start()
            ry.append(r)
            r = rdma(src, src, sz, rz_sem, i, partner_z)
            r.start()
            rz.append(r)

        for k, (j, s, rows) in enumerate(RELAY_Y):
            rz[j].wait_recv()
            src = out_ref.at[pl.ds(rem + qz + s, rows), :]
            r = rdma(src, src, sy, ry_sem, NCH + k, partner_y)
            r.start()
            ry.append(r)
        for k, (j, s, rows) in enumerate(RELAY_Z):
            ry[j].wait_recv()
            src = out_ref.at[pl.ds(rem + qy + s, rows), :]
            r = rdma(src, src, sz, rz_sem, NCH + k, partner_z)
            r.start()
            rz.append(r)

        for r in rx + ry + rz:
            r.wait_send()
        rx[NCH].wait_recv()
        waited_y = {j for j, _, _ in RELAY_Z}
        waited_z = {j for j, _, _ in RELAY_Y}
        for i in range(len(ry)):
            if i not in waited_y:
                ry[i].wait_recv()
        for i in range(len(rz)):
            if i not in waited_z:
                rz[i].wait_recv()

    n_y = NCH + len(RELAY_Y)
    n_z = NCH + len(RELAY_Z)
    return pl.pallas_call(
        body,
        out_shape=jax.ShapeDtypeStruct((2 * m_per, n), x.dtype),
        in_specs=[pl.BlockSpec(memory_space=pltpu.VMEM)],
        out_specs=pl.BlockSpec(memory_space=pltpu.VMEM),
        scratch_shapes=[
            pltpu.SemaphoreType.DMA((NCH + 1,)),
            pltpu.SemaphoreType.DMA((NCH + 1,)),
            pltpu.SemaphoreType.DMA((n_y,)),
            pltpu.SemaphoreType.DMA((n_y,)),
            pltpu.SemaphoreType.DMA((n_z,)),
            pltpu.SemaphoreType.DMA((n_z,)),
        ],
        compiler_params=pltpu.CompilerParams(collective_id=0),
    )(x)


# device time: 19081 ns/iter; 1.5488x vs baseline; 1.0152x over previous
import jax
import jax.numpy as jnp
from jax import lax
from jax.experimental import pallas as pl
from jax.experimental.pallas import tpu as pltpu

QR = 256
CH = 32
NCH = QR // CH

RELAY_Y = ((2, 80, 16), (3, 96, 32), (4, 128, 32), (5, 160, 8))
RELAY_Z = ((5, 168, 24), (6, 192, 32), (7, 224, 32))
DX_ROWS = 80

MESH = pl.DeviceIdType.MESH


def kernel(x):
    m_per, n = x.shape

    def body(x_ref, out_ref, sx, rx_sem, sy, ry_sem, sz, rz_sem, loc_sem):
        my_x = lax.axis_index("x")
        my_y = lax.axis_index("y")
        my_z = lax.axis_index("z")
        other_x = 1 - my_x
        p = my_y % 2
        q = my_z % 2
        partner_x = (other_x, my_y, my_z)
        partner_y = (my_x, my_y ^ 1, my_z)
        partner_z = (my_x, my_y, my_z ^ 1)

        qown = (2 * p + q) * QR
        qy = (2 * (1 - p) + q) * QR
        qz = (2 * p + (1 - q)) * QR
        qd = (2 * (1 - p) + (1 - q)) * QR
        own = my_x * m_per
        rem = other_x * m_per

        barrier_sem = pltpu.get_barrier_semaphore()
        for nbr in (partner_x, partner_y, partner_z):
            pl.semaphore_signal(
                barrier_sem, inc=1, device_id=nbr, device_id_type=MESH,
            )
        pl.semaphore_wait(barrier_sem, 3)

        def rdma(src, dst, send_sems, recv_sems, i, dev):
            return pltpu.make_async_remote_copy(
                src_ref=src, dst_ref=dst,
                send_sem=send_sems.at[i], recv_sem=recv_sems.at[i],
                device_id=dev, device_id_type=MESH,
            )

        rx = []
        for i in range(NCH):
            r = rdma(
                x_ref.at[pl.ds(qown + i * CH, CH), :],
                out_ref.at[pl.ds(own + qown + i * CH, CH), :],
                sx, rx_sem, i, partner_x,
            )
            r.start()
            rx.append(r)
        r = rdma(
            x_ref.at[pl.ds(qd, DX_ROWS), :],
            out_ref.at[pl.ds(own + qd, DX_ROWS), :],
            sx, rx_sem, NCH, partner_x,
        )
        r.start()
        rx.append(r)

        own_copy = pltpu.make_async_copy(
            x_ref, out_ref.at[pl.ds(own, m_per), :], loc_sem
        )
        own_copy.start()

        ry, rz = [], []
        for i in range(NCH):
            rx[i].wait_recv()
            src = out_ref.at[pl.ds(rem + qown + i * CH, CH), :]
            r = rdma(src, src, sy, ry_sem, i, partner_y)
            r.start()
            ry.append(r)
            r = rdma(src, src, sz, rz_sem, i, partner_z)
            r.start()
            rz.append(r)

        for k, (j, s, rows) in enumerate(RELAY_Y):
            rz[j].wait_recv()
            src = out_ref.at[pl.ds(rem + qz + s, rows), :]
            r = rdma(src, src, sy, ry_sem, NCH + k, partner_y)
            r.start()
            ry.append(r)
        for k, (j, s, rows) in enumerate(RELAY_Z):
            ry[j].wait_recv()
            src = out_ref.at[pl.ds(rem + qy + s, rows), :]
            r = rdma(src, src, sz, rz_sem, NCH + k, partner_z)
            r.start()
            rz.append(r)

        for r in rx + ry + rz:
            r.wait_send()
        rx[NCH].wait_recv()
        waited_y = {j for j, _, _ in RELAY_Z}
        waited_z = {j for j, _, _ in RELAY_Y}
        for i in range(len(ry)):
            if i not in waited_y:
                ry[i].wait_recv()
        for i in range(len(rz)):
            if i not in waited_z:
                rz[i].wait_recv()
        own_copy.wait()

    n_y = NCH + len(RELAY_Y)
    n_z = NCH + len(RELAY_Z)
    return pl.pallas_call(
        body,
        out_shape=jax.ShapeDtypeStruct((2 * m_per, n), x.dtype),
        in_specs=[pl.BlockSpec(memory_space=pl.ANY)],
        out_specs=pl.BlockSpec(memory_space=pl.ANY),
        scratch_shapes=[
            pltpu.SemaphoreType.DMA((NCH + 1,)),
            pltpu.SemaphoreType.DMA((NCH + 1,)),
            pltpu.SemaphoreType.DMA((n_y,)),
            pltpu.SemaphoreType.DMA((n_y,)),
            pltpu.SemaphoreType.DMA((n_z,)),
            pltpu.SemaphoreType.DMA((n_z,)),
            pltpu.SemaphoreType.DMA,
        ],
        compiler_params=pltpu.CompilerParams(collective_id=0),
    )(x)
